# baseline (speedup 1.0000x reference)
import functools

import jax
import jax.numpy as jnp
from jax import lax
from jax.experimental import pallas as pl
from jax.experimental.pallas import tpu as pltpu
from jax.experimental.pallas import tpu_sc as plsc

NUM_FIELDS = 26
VOCAB = 100000
EMBED = 16
B = 16384
MLP_IN = NUM_FIELDS * EMBED
H1, H2 = 128, 64

BF = B * NUM_FIELDS
NW = 32
PER_W = BF // NW
CROWS = 128
NCH = PER_W // CROWS


def _sc_gather(flat_idx, table_e, table_lv):
    mesh = plsc.VectorSubcoreMesh(core_axis_name="c", subcore_axis_name="s")

    @functools.partial(
        pl.kernel,
        mesh=mesh,
        compiler_params=pltpu.CompilerParams(
            use_tc_tiling_on_sc=False, needs_layout_passes=False),
        out_type=[
            jax.ShapeDtypeStruct((BF, EMBED), jnp.float32),
            jax.ShapeDtypeStruct((BF,), jnp.float32),
        ],
        scratch_types=[
            pltpu.VMEM((NCH, CROWS), jnp.int32),
            pltpu.VMEM((CROWS,), jnp.int32),
            pltpu.VMEM((CROWS, EMBED), jnp.float32),
            pltpu.VMEM((CROWS, EMBED), jnp.float32),
            pltpu.VMEM((CROWS,), jnp.float32),
            pltpu.SemaphoreType.DMA,
            pltpu.SemaphoreType.DMA,
        ],
    )
    def k(idx_hbm, te_hbm, tlv_hbm, oute_hbm, outl_hbm, idx_v, hi_v, e_buf,
          l_buf, l_sel, sem_e, sem_l):
        wid = lax.axis_index("s") * 2 + lax.axis_index("c")
        base = wid * PER_W
        pltpu.sync_copy(idx_hbm.at[wid], idx_v)
        grp = CROWS // 16

        def body(j, _):
            for g in range(grp):
                idxs = idx_v[j, pl.ds(g * 16, 16)]
                hi_v[pl.ds(g * 16, 16)] = idxs >> 4
            ce = pltpu.async_copy(te_hbm.at[idx_v.at[j]], e_buf, sem_e)
            cl = pltpu.async_copy(tlv_hbm.at[hi_v], l_buf, sem_l)
            ce.wait()
            cl.wait()
            for g in range(grp):
                idxs = idx_v[j, pl.ds(g * 16, 16)]
                rows = lax.iota(jnp.int32, 16) + g * 16
                l_sel[pl.ds(g * 16, 16)] = plsc.load_gather(
                    l_buf, [rows, idxs & 15])
            pltpu.sync_copy(e_buf, oute_hbm.at[pl.ds(base + j * CROWS, CROWS)])
            pltpu.sync_copy(l_sel, outl_hbm.at[pl.ds(base + j * CROWS, CROWS)])
            return 0

        lax.fori_loop(0, NCH, body, 0)

    return k(flat_idx, table_e, table_lv)


def _tc_body(e_ref, lin_ref, w1_ref, b1_ref, w2_ref, b2_ref, w3_ref, b3_ref,
             s_ref, out_ref):
    e = e_ref[...]
    h = jnp.maximum(
        jnp.dot(e, w1_ref[...], preferred_element_type=jnp.float32)
        + b1_ref[...], 0.0)
    h = jnp.maximum(
        jnp.dot(h, w2_ref[...], preferred_element_type=jnp.float32)
        + b2_ref[...], 0.0)
    dnn = jnp.sum(h * w3_ref[...], axis=1, keepdims=True) + b3_ref[...]
    s = s_ref[...]
    se = jnp.dot(e, s, preferred_element_type=jnp.float32)
    ss = jnp.dot(e * e, s, preferred_element_type=jnp.float32)
    fm = 0.5 * jnp.sum(se * se - ss, axis=1, keepdims=True)
    linear = jnp.sum(lin_ref[...], axis=1, keepdims=True)
    out_ref[...] = jax.nn.sigmoid(dnn + fm + linear)


def _tc_forward(e, lin, W1, b1, W2, b2, w3r, b3):
    BT = 2048
    grid = B // BT
    s_mat = jnp.tile(jnp.eye(EMBED, dtype=jnp.float32), (NUM_FIELDS, 1))
    return pl.pallas_call(
        _tc_body,
        grid=(grid,),
        in_specs=[
            pl.BlockSpec((BT, MLP_IN), lambda i: (i, 0)),
            pl.BlockSpec((BT, NUM_FIELDS), lambda i: (i, 0)),
            pl.BlockSpec((MLP_IN, H1), lambda i: (0, 0)),
            pl.BlockSpec((1, H1), lambda i: (0, 0)),
            pl.BlockSpec((H1, H2), lambda i: (0, 0)),
            pl.BlockSpec((1, H2), lambda i: (0, 0)),
            pl.BlockSpec((1, H2), lambda i: (0, 0)),
            pl.BlockSpec((1, 1), lambda i: (0, 0)),
            pl.BlockSpec((MLP_IN, EMBED), lambda i: (0, 0)),
        ],
        out_specs=pl.BlockSpec((BT, 1), lambda i: (i, 0)),
        out_shape=jax.ShapeDtypeStruct((B, 1), jnp.float32),
    )(e, lin, W1, b1, W2, b2, w3r, b3, s_mat)


def kernel(x, table_e, table_l, W1, b1, W2, b2, W3, b3):
    offsets = jnp.arange(NUM_FIELDS, dtype=jnp.int32) * VOCAB
    flat_idx = (x.astype(jnp.int32) + offsets[None, :]).reshape(NW, NCH, CROWS)
    table_lv = table_l.reshape(NUM_FIELDS * VOCAB // EMBED, EMBED)
    embed, lin = _sc_gather(flat_idx, table_e, table_lv)
    out = _tc_forward(
        embed.reshape(B, MLP_IN),
        lin.reshape(B, NUM_FIELDS),
        W1, b1.reshape(1, H1), W2, b2.reshape(1, H2), W3.reshape(1, H2),
        b3.reshape(1, 1),
    )
    return out


# speedup vs baseline: 1.0675x; 1.0675x over previous
import functools

import jax
import jax.numpy as jnp
from jax import lax
from jax.experimental import pallas as pl
from jax.experimental.pallas import tpu as pltpu
from jax.experimental.pallas import tpu_sc as plsc

NUM_FIELDS = 26
VOCAB = 100000
EMBED = 16
B = 16384
MLP_IN = NUM_FIELDS * EMBED
H1, H2 = 128, 64

BF = B * NUM_FIELDS
NW = 32
PER_W = BF // NW
KC = 13
CROWS = KC * 128
NCH = PER_W // CROWS


def _sc_gather(flat_idx, table_e, table_l):
    mesh = plsc.VectorSubcoreMesh(core_axis_name="c", subcore_axis_name="s")

    @functools.partial(
        pl.kernel,
        mesh=mesh,
        compiler_params=pltpu.CompilerParams(
            use_tc_tiling_on_sc=False, needs_layout_passes=False),
        out_type=[
            jax.ShapeDtypeStruct((NW * NCH, CROWS, EMBED), jnp.float32),
            jax.ShapeDtypeStruct((NW * NCH, CROWS), jnp.float32),
        ],
        scratch_types=[
            pltpu.VMEM((NCH, CROWS), jnp.int32),
            pltpu.VMEM((2, CROWS, EMBED), jnp.float32),
            pltpu.VMEM((2, CROWS), jnp.float32),
            pltpu.SemaphoreType.DMA,
            pltpu.SemaphoreType.DMA,
            pltpu.SemaphoreType.DMA,
            pltpu.SemaphoreType.DMA,
        ],
    )
    def k(idx_hbm, te_hbm, tl_hbm, oute_hbm, outl_hbm, idx_v, e_buf, l_buf,
          sem_e0, sem_e1, sem_l0, sem_l1):
        wid = lax.axis_index("s") * 2 + lax.axis_index("c")
        base = wid * NCH
        pltpu.sync_copy(idx_hbm.at[wid], idx_v)
        sem_e = (sem_e0, sem_e1)
        sem_l = (sem_l0, sem_l1)
        prev = None
        for j in range(NCH):
            s = j % 2
            ce = pltpu.async_copy(te_hbm.at[idx_v.at[j]], e_buf.at[s],
                                  sem_e[s])
            cl = pltpu.async_copy(tl_hbm.at[idx_v.at[j]], l_buf.at[s],
                                  sem_l[s])
            if prev is not None:
                pj, pce, pcl = prev
                pce.wait()
                pcl.wait()
                pltpu.sync_copy(e_buf.at[pj % 2], oute_hbm.at[base + pj])
                pltpu.sync_copy(l_buf.at[pj % 2], outl_hbm.at[base + pj])
            prev = (j, ce, cl)
        pj, pce, pcl = prev
        pce.wait()
        pcl.wait()
        pltpu.sync_copy(e_buf.at[pj % 2], oute_hbm.at[base + pj])
        pltpu.sync_copy(l_buf.at[pj % 2], outl_hbm.at[base + pj])

    return k(flat_idx, table_e, table_l)


def _tc_body(e_ref, lin_ref, w1_ref, b1_ref, w2_ref, b2_ref, w3_ref, b3_ref,
             s_ref, out_ref):
    e = e_ref[...]
    h = jnp.maximum(
        jnp.dot(e, w1_ref[...], preferred_element_type=jnp.float32)
        + b1_ref[...], 0.0)
    h = jnp.maximum(
        jnp.dot(h, w2_ref[...], preferred_element_type=jnp.float32)
        + b2_ref[...], 0.0)
    dnn = jnp.sum(h * w3_ref[...], axis=1, keepdims=True) + b3_ref[...]
    s = s_ref[...]
    se = jnp.dot(e, s, preferred_element_type=jnp.float32)
    ss = jnp.dot(e * e, s, preferred_element_type=jnp.float32)
    fm = 0.5 * jnp.sum(se * se - ss, axis=1, keepdims=True)
    linear = jnp.sum(lin_ref[...], axis=1, keepdims=True)
    out_ref[...] = jax.nn.sigmoid(dnn + fm + linear)


def _tc_forward(e, lin, W1, b1, W2, b2, w3r, b3):
    BT = 2048
    grid = B // BT
    s_mat = jnp.tile(jnp.eye(EMBED, dtype=jnp.float32), (NUM_FIELDS, 1))
    return pl.pallas_call(
        _tc_body,
        grid=(grid,),
        in_specs=[
            pl.BlockSpec((BT, MLP_IN), lambda i: (i, 0)),
            pl.BlockSpec((BT, NUM_FIELDS), lambda i: (i, 0)),
            pl.BlockSpec((MLP_IN, H1), lambda i: (0, 0)),
            pl.BlockSpec((1, H1), lambda i: (0, 0)),
            pl.BlockSpec((H1, H2), lambda i: (0, 0)),
            pl.BlockSpec((1, H2), lambda i: (0, 0)),
            pl.BlockSpec((1, H2), lambda i: (0, 0)),
            pl.BlockSpec((1, 1), lambda i: (0, 0)),
            pl.BlockSpec((MLP_IN, EMBED), lambda i: (0, 0)),
        ],
        out_specs=pl.BlockSpec((BT, 1), lambda i: (i, 0)),
        out_shape=jax.ShapeDtypeStruct((B, 1), jnp.float32),
    )(e, lin, W1, b1, W2, b2, w3r, b3, s_mat)


def kernel(x, table_e, table_l, W1, b1, W2, b2, W3, b3):
    offsets = jnp.arange(NUM_FIELDS, dtype=jnp.int32) * VOCAB
    flat_idx = (x.astype(jnp.int32) + offsets[None, :]).reshape(NW, NCH, CROWS)
    embed, lin = _sc_gather(flat_idx, table_e,
                            table_l.reshape(NUM_FIELDS * VOCAB))
    out = _tc_forward(
        embed.reshape(B, MLP_IN),
        lin.reshape(B, NUM_FIELDS),
        W1, b1.reshape(1, H1), W2, b2.reshape(1, H2), W3.reshape(1, H2),
        b3.reshape(1, 1),
    )
    return out
